# baseline (speedup 1.0000x reference)
import functools

import jax
import jax.numpy as jnp
from jax import lax
from jax.experimental import pallas as pl
from jax.experimental.pallas import tpu as pltpu
from jax.experimental.pallas import tpu_sc as plsc

N = 10000
CH = 128
NCORE = 2
NSUB = 16
NP = 10112
SPR = NP // NSUB
CK = 128
K = 80
CAP = NCORE * NSUB * K * CK
DUMP = N

NBUF = 4
GC = 88
KP = 8
NPASS = 15
GT = NPASS * KP
CAP2 = NCORE * NSUB * GT * GC
GP = N + 8
AP = N
ASPR = 632
ASPR_L = AP - 15 * ASPR

_mesh = plsc.VectorSubcoreMesh(core_axis_name="c", subcore_axis_name="s")


@functools.partial(
    pl.kernel,
    out_type=jax.ShapeDtypeStruct((NCORE, NP, CH), jnp.float32),
    mesh=_mesh,
    scratch_types=[
        pltpu.VMEM((K, CK), jnp.int32),
        pltpu.VMEM((CK, CH), jnp.float32),
        pltpu.VMEM_SHARED((NP, CH), jnp.float32),
    ],
)
def _deg_kernel(col_hbm, ones_hbm, zeros_hbm, out_hbm, colv, onesv, deg_sh):
    cid = lax.axis_index("c")
    sid = lax.axis_index("s")
    base = sid * SPR
    pltpu.sync_copy(zeros_hbm.at[pl.ds(base, SPR)], deg_sh.at[pl.ds(base, SPR)])
    pltpu.sync_copy(col_hbm.at[cid, sid], colv)
    pltpu.sync_copy(ones_hbm, onesv)
    plsc.subcore_barrier()

    def body(k, carry):
        pltpu.sync_copy(onesv, deg_sh.at[colv.at[k]], add=True)
        return carry

    lax.fori_loop(0, K, body, 0)
    plsc.subcore_barrier()
    pltpu.sync_copy(deg_sh.at[pl.ds(base, SPR)], out_hbm.at[cid, pl.ds(base, SPR)])


@functools.partial(
    pl.kernel,
    out_type=jax.ShapeDtypeStruct((NCORE, AP, CH), jnp.float32),
    mesh=_mesh,
    scratch_types=[
        pltpu.VMEM((2, 2 * KP, GC), jnp.int32),
        pltpu.VMEM((NBUF, GC, CH), jnp.float32),
        pltpu.VMEM_SHARED((AP, CH), jnp.float32),
        pltpu.SemaphoreType.DMA,
        pltpu.SemaphoreType.DMA,
        pltpu.SemaphoreType.DMA,
        pltpu.SemaphoreType.DMA,
        pltpu.SemaphoreType.DMA,
    ],
)
def _scat_kernel(idx_hbm, g_hbm, zeros_hbm, out_hbm,
                 idxv, bufs, acc_sh, sem0, sem1, sem2, sem3, semi):
    cid = lax.axis_index("c")
    sid = lax.axis_index("s")
    base = sid * ASPR
    sems = (sem0, sem1, sem2, sem3)

    @pl.when(sid < NSUB - 1)
    def _():
        pltpu.sync_copy(zeros_hbm.at[pl.ds(0, ASPR)],
                        acc_sh.at[pl.ds(base, ASPR)])

    @pl.when(sid == NSUB - 1)
    def _():
        pltpu.sync_copy(zeros_hbm.at[pl.ds(0, ASPR_L)],
                        acc_sh.at[pl.ds(base, ASPR_L)])

    plsc.subcore_barrier()

    pltpu.sync_copy(idx_hbm.at[cid, sid, 0], idxv.at[0])
    for t in range(NBUF):
        pltpu.async_copy(g_hbm.at[idxv.at[0, t]], bufs.at[t], sems[t])

    for gk in range(GT):
        p, k = divmod(gk, KP)
        s = p % 2
        if k == 0 and p + 1 < NPASS:
            pltpu.async_copy(idx_hbm.at[cid, sid, p + 1],
                             idxv.at[(p + 1) % 2], semi)
        t = gk % NBUF
        pltpu.make_async_copy(
            g_hbm.at[idxv.at[s, k]], bufs.at[t], sems[t]).wait()
        pltpu.sync_copy(bufs.at[t], acc_sh.at[idxv.at[s, KP + k]], add=True)
        nk = gk + NBUF
        if nk < GT:
            np_, kn = divmod(nk, KP)
            sn = np_ % 2
            if kn == 0:
                pltpu.make_async_copy(idx_hbm.at[cid, sid, np_],
                                      idxv.at[sn], semi).wait()
            pltpu.async_copy(g_hbm.at[idxv.at[sn, kn]], bufs.at[t], sems[t])

    plsc.subcore_barrier()

    @pl.when(sid < NSUB - 1)
    def _():
        pltpu.sync_copy(acc_sh.at[pl.ds(base, ASPR)],
                        out_hbm.at[cid, pl.ds(base, ASPR)])

    @pl.when(sid == NSUB - 1)
    def _():
        pltpu.sync_copy(acc_sh.at[pl.ds(base, ASPR_L)],
                        out_hbm.at[cid, pl.ds(base, ASPR_L)])


def _tc_transform(x, W, dp0, dp1):
    def body(x_ref, w_ref, d0_ref, d1_ref, g_ref, dinv_ref):
        deg = d0_ref[...] + d1_ref[...] + 1.0
        dinv = lax.rsqrt(deg)
        h = jnp.dot(x_ref[...], w_ref[...], preferred_element_type=jnp.float32)
        g_ref[:N, :] = h * dinv
        g_ref[N:, :] = jnp.zeros((GP - N, CH), jnp.float32)
        dinv_ref[...] = dinv

    return pl.pallas_call(
        body,
        out_shape=(
            jax.ShapeDtypeStruct((GP, CH), jnp.float32),
            jax.ShapeDtypeStruct((N, 1), jnp.float32),
        ),
    )(x, W, dp0, dp1)


def _tc_combine(acc, g, dinv, b2):
    def body(a_ref, g_ref, di_ref, b_ref, o_ref):
        s = a_ref[0] + a_ref[1] + g_ref[:N, :]
        o_ref[...] = s * di_ref[...] + b_ref[...]

    return pl.pallas_call(
        body,
        out_shape=jax.ShapeDtypeStruct((N, CH), jnp.float32),
    )(acc, g, dinv, b2)


def kernel(x, edge_index, W, b):
    row = edge_index[0].astype(jnp.int32)
    col = edge_index[1].astype(jnp.int32)
    pad = CAP - row.shape[0]
    row_p = jnp.concatenate([row, jnp.zeros((pad,), jnp.int32)])
    col_p = jnp.concatenate([col, jnp.full((pad,), DUMP, jnp.int32)])
    col_deg = col_p.reshape(NCORE, NSUB, K, CK)

    pad2 = CAP2 - row.shape[0]
    row_s = jnp.concatenate([row, jnp.full((pad2,), N, jnp.int32)])
    col_s = jnp.concatenate([col, jnp.zeros((pad2,), jnp.int32)])
    row_r = row_s.reshape(NCORE, NSUB, NPASS, KP, GC)
    col_r = col_s.reshape(NCORE, NSUB, NPASS, KP, GC)
    idx = jnp.concatenate([row_r, col_r], axis=3)

    ones_rows = jnp.ones((CK, CH), jnp.float32)
    zbig = jnp.zeros((NP, CH), jnp.float32)

    deg_part = _deg_kernel(col_deg, ones_rows, zbig)
    dp0 = deg_part[0, :N, 0:1]
    dp1 = deg_part[1, :N, 0:1]
    g, dinv = _tc_transform(x, W, dp0, dp1)
    acc = _scat_kernel(idx, g, zbig)
    return _tc_combine(acc, g, dinv, b.reshape(1, CH))


# speedup vs baseline: 1.4862x; 1.4862x over previous
import functools

import jax
import jax.numpy as jnp
from jax import lax
from jax.experimental import pallas as pl
from jax.experimental.pallas import tpu as pltpu
from jax.experimental.pallas import tpu_sc as plsc

N = 10000
CH = 128
NCORE = 2
NSUB = 16
NP = 10112
SPR = NP // NSUB
CK = 128
K = 80
CAP = NCORE * NSUB * K * CK
DUMP = N

_mesh = plsc.VectorSubcoreMesh(core_axis_name="c", subcore_axis_name="s")


@functools.partial(
    pl.kernel,
    out_type=jax.ShapeDtypeStruct((NCORE, NP, CH), jnp.float32),
    mesh=_mesh,
    scratch_types=[
        pltpu.VMEM((K, CK), jnp.int32),
        pltpu.VMEM((CK, CH), jnp.float32),
        pltpu.VMEM_SHARED((NP, CH), jnp.float32),
    ],
)
def _deg_kernel(col_hbm, ones_hbm, zeros_hbm, out_hbm, colv, onesv, deg_sh):
    cid = lax.axis_index("c")
    sid = lax.axis_index("s")
    base = sid * SPR
    pltpu.sync_copy(zeros_hbm.at[pl.ds(base, SPR)], deg_sh.at[pl.ds(base, SPR)])
    pltpu.sync_copy(col_hbm.at[cid, sid], colv)
    pltpu.sync_copy(ones_hbm, onesv)
    plsc.subcore_barrier()

    def body(k, carry):
        pltpu.sync_copy(onesv, deg_sh.at[colv.at[k]], add=True)
        return carry

    lax.fori_loop(0, K, body, 0)
    plsc.subcore_barrier()
    pltpu.sync_copy(deg_sh.at[pl.ds(base, SPR)], out_hbm.at[cid, pl.ds(base, SPR)])


@functools.partial(
    pl.kernel,
    out_type=jax.ShapeDtypeStruct((NCORE, NP, CH), jnp.float32),
    mesh=_mesh,
    scratch_types=[
        pltpu.VMEM((K, CK), jnp.int32),
        pltpu.VMEM((K, CK), jnp.int32),
        pltpu.VMEM((CK, CH), jnp.float32),
        pltpu.VMEM_SHARED((NP, CH), jnp.float32),
    ],
)
def _scat_kernel(row_hbm, col_hbm, g_hbm, zeros_hbm, out_hbm,
                 rowv, colv, buf, acc_sh):
    cid = lax.axis_index("c")
    sid = lax.axis_index("s")
    base = sid * SPR
    pltpu.sync_copy(zeros_hbm.at[pl.ds(base, SPR)], acc_sh.at[pl.ds(base, SPR)])
    pltpu.sync_copy(row_hbm.at[cid, sid], rowv)
    pltpu.sync_copy(col_hbm.at[cid, sid], colv)
    plsc.subcore_barrier()

    def body(k, carry):
        pltpu.sync_copy(g_hbm.at[rowv.at[k]], buf)
        pltpu.sync_copy(buf, acc_sh.at[colv.at[k]], add=True)
        return carry

    lax.fori_loop(0, K, body, 0)
    plsc.subcore_barrier()
    pltpu.sync_copy(acc_sh.at[pl.ds(base, SPR)], out_hbm.at[cid, pl.ds(base, SPR)])


def _tc_transform(x, W, dp0, dp1):
    def body(x_ref, w_ref, d0_ref, d1_ref, g_ref, dinv_ref):
        deg = d0_ref[...] + d1_ref[...] + 1.0
        dinv = lax.rsqrt(deg)
        h = jnp.dot(x_ref[...], w_ref[...], preferred_element_type=jnp.float32)
        g_ref[...] = h * dinv
        dinv_ref[...] = dinv

    return pl.pallas_call(
        body,
        out_shape=(
            jax.ShapeDtypeStruct((N, CH), jnp.float32),
            jax.ShapeDtypeStruct((N, 1), jnp.float32),
        ),
    )(x, W, dp0, dp1)


def _tc_combine(acc, g, dinv, b2):
    def body(a_ref, g_ref, di_ref, b_ref, o_ref):
        s = a_ref[0, :N, :] + a_ref[1, :N, :] + g_ref[...]
        o_ref[...] = s * di_ref[...] + b_ref[...]

    return pl.pallas_call(
        body,
        out_shape=jax.ShapeDtypeStruct((N, CH), jnp.float32),
    )(acc, g, dinv, b2)


def kernel(x, edge_index, W, b):
    row = edge_index[0].astype(jnp.int32)
    col = edge_index[1].astype(jnp.int32)
    pad = CAP - row.shape[0]
    row_p = jnp.concatenate([row, jnp.zeros((pad,), jnp.int32)])
    col_p = jnp.concatenate([col, jnp.full((pad,), DUMP, jnp.int32)])
    row_r = row_p.reshape(NCORE, NSUB, K, CK)
    col_r = col_p.reshape(NCORE, NSUB, K, CK)

    ones_rows = jnp.ones((CK, CH), jnp.float32)
    zbig = jnp.zeros((NP, CH), jnp.float32)

    deg_part = _deg_kernel(col_r, ones_rows, zbig)
    dp0 = deg_part[0, :N, 0:1]
    dp1 = deg_part[1, :N, 0:1]
    g, dinv = _tc_transform(x, W, dp0, dp1)
    acc = _scat_kernel(row_r, col_r, g, zbig)
    return _tc_combine(acc, g, dinv, b.reshape(1, CH))


# speedup vs baseline: 1.5675x; 1.0547x over previous
import functools

import jax
import jax.numpy as jnp
from jax import lax
from jax.experimental import pallas as pl
from jax.experimental.pallas import tpu as pltpu
from jax.experimental.pallas import tpu_sc as plsc

N = 10000
CH = 128
NCORE = 2
NSUB = 16
NP = 10112
SPR = NP // NSUB
CK = 128
K = 80
K2 = 40
CAP = NCORE * NSUB * K * CK
DUMP = N

_mesh = plsc.VectorSubcoreMesh(core_axis_name="c", subcore_axis_name="s")


@functools.partial(
    pl.kernel,
    out_type=jax.ShapeDtypeStruct((NCORE, NP, CH), jnp.float32),
    mesh=_mesh,
    scratch_types=[
        pltpu.VMEM((K, CK), jnp.int32),
        pltpu.VMEM((CK, CH), jnp.float32),
        pltpu.VMEM_SHARED((NP, CH), jnp.float32),
    ],
)
def _deg_kernel(col_hbm, ones_hbm, zeros_hbm, out_hbm, colv, onesv, deg_sh):
    cid = lax.axis_index("c")
    sid = lax.axis_index("s")
    base = sid * SPR
    pltpu.sync_copy(zeros_hbm.at[pl.ds(base, SPR)], deg_sh.at[pl.ds(base, SPR)])
    pltpu.sync_copy(col_hbm.at[cid, sid], colv)
    pltpu.sync_copy(ones_hbm, onesv)
    plsc.subcore_barrier()

    def body(k, carry):
        pltpu.sync_copy(onesv, deg_sh.at[colv.at[k]], add=True)
        return carry

    lax.fori_loop(0, K, body, 0)
    plsc.subcore_barrier()
    pltpu.sync_copy(deg_sh.at[pl.ds(base, SPR)], out_hbm.at[cid, pl.ds(base, SPR)])


@functools.partial(
    pl.kernel,
    out_type=jax.ShapeDtypeStruct((NCORE, NP, CH), jnp.float32),
    mesh=_mesh,
    scratch_types=[
        pltpu.VMEM((K2, CK), jnp.int32),
        pltpu.VMEM((K2, CK), jnp.int32),
        pltpu.VMEM((2, CK, CH), jnp.float32),
        pltpu.VMEM_SHARED((NP, CH), jnp.float32),
        pltpu.SemaphoreType.DMA,
        pltpu.SemaphoreType.DMA,
    ],
)
def _scat_kernel(row_hbm, col_hbm, g_hbm, zeros_hbm, out_hbm,
                 rowv, colv, bufs, acc_sh, sem0, sem1):
    cid = lax.axis_index("c")
    sid = lax.axis_index("s")
    base = sid * SPR
    sems = (sem0, sem1)
    pltpu.sync_copy(zeros_hbm.at[pl.ds(base, SPR)], acc_sh.at[pl.ds(base, SPR)])
    plsc.subcore_barrier()

    for h in range(K // K2):
        pltpu.sync_copy(row_hbm.at[cid, sid, h], rowv)
        pltpu.sync_copy(col_hbm.at[cid, sid, h], colv)
        pltpu.async_copy(g_hbm.at[rowv.at[0]], bufs.at[0], sem0)
        for k in range(K2):
            t = k % 2
            pltpu.make_async_copy(
                g_hbm.at[rowv.at[k]], bufs.at[t], sems[t]).wait()
            if k + 1 < K2:
                pltpu.async_copy(g_hbm.at[rowv.at[k + 1]], bufs.at[1 - t],
                                 sems[1 - t])
            pltpu.sync_copy(bufs.at[t], acc_sh.at[colv.at[k]], add=True)

    plsc.subcore_barrier()
    pltpu.sync_copy(acc_sh.at[pl.ds(base, SPR)], out_hbm.at[cid, pl.ds(base, SPR)])


def _tc_transform(x, W, dp0, dp1):
    def body(x_ref, w_ref, d0_ref, d1_ref, g_ref, dinv_ref):
        deg = d0_ref[...] + d1_ref[...] + 1.0
        dinv = lax.rsqrt(deg)
        h = jnp.dot(x_ref[...], w_ref[...], preferred_element_type=jnp.float32)
        g_ref[...] = h * dinv
        dinv_ref[...] = dinv

    return pl.pallas_call(
        body,
        out_shape=(
            jax.ShapeDtypeStruct((N, CH), jnp.float32),
            jax.ShapeDtypeStruct((N, 1), jnp.float32),
        ),
    )(x, W, dp0, dp1)


def _tc_combine(acc, g, dinv, b2):
    def body(a_ref, g_ref, di_ref, b_ref, o_ref):
        s = a_ref[0, :N, :] + a_ref[1, :N, :] + g_ref[...]
        o_ref[...] = s * di_ref[...] + b_ref[...]

    return pl.pallas_call(
        body,
        out_shape=jax.ShapeDtypeStruct((N, CH), jnp.float32),
    )(acc, g, dinv, b2)


def kernel(x, edge_index, W, b):
    row = edge_index[0].astype(jnp.int32)
    col = edge_index[1].astype(jnp.int32)
    pad = CAP - row.shape[0]
    row_p = jnp.concatenate([row, jnp.zeros((pad,), jnp.int32)])
    col_p = jnp.concatenate([col, jnp.full((pad,), DUMP, jnp.int32)])
    row_r = row_p.reshape(NCORE, NSUB, K, CK)
    col_r = col_p.reshape(NCORE, NSUB, K, CK)

    ones_rows = jnp.ones((CK, CH), jnp.float32)
    zbig = jnp.zeros((NP, CH), jnp.float32)

    row_r2 = row_p.reshape(NCORE, NSUB, K // K2, K2, CK)
    col_r2 = col_p.reshape(NCORE, NSUB, K // K2, K2, CK)

    deg_part = _deg_kernel(col_r, ones_rows, zbig)
    dp0 = deg_part[0, :N, 0:1]
    dp1 = deg_part[1, :N, 0:1]
    g, dinv = _tc_transform(x, W, dp0, dp1)
    acc = _scat_kernel(row_r2, col_r2, g, zbig)
    return _tc_combine(acc, g, dinv, b.reshape(1, CH))
